# speedup vs baseline: 28.3115x; 28.3115x over previous
import functools

import jax
import jax.numpy as jnp
from jax.experimental import pallas as pl
from jax.experimental.pallas import tpu as pltpu

_B = 64
_N = 20000
_RG = 8
_RATIO = 3


def _body(d_ref, t_ref, out_ref, keys_ref, acc_ref):
    g = pl.program_id(0)

    @pl.when(g == 0)
    def _init():
        acc_ref[0] = 0.0
        acc_ref[1] = 0.0

    d = d_ref[...]
    lab = t_ref[...]
    pos = lab > 0

    loss = jnp.maximum(d, 0.0) + jnp.log1p(jnp.exp(-jnp.abs(d)))
    ce_pos = loss - d

    num_pos = jnp.sum(pos.astype(jnp.int32), axis=1, keepdims=True)
    sum_pos_ce = jnp.sum(jnp.where(pos, ce_pos, 0.0))
    sum_neg_row = jnp.sum(jnp.where(pos, 0.0, loss), axis=1, keepdims=True)

    keys = jnp.where(pos, jnp.int32(-1),
                     jax.lax.bitcast_convert_type(loss, jnp.int32))
    keys_ref[...] = keys

    npc = jnp.maximum(num_pos, 1)
    num_neg = _N - num_pos
    num_sel = jnp.minimum(npc * _RATIO, num_neg)
    ks = jnp.maximum(num_sel, 1)

    def step(i, prefix):
        t = prefix | (jnp.int32(1) << (30 - i))
        cnt = jnp.sum((keys_ref[...] >= t).astype(jnp.int32), axis=1,
                      keepdims=True)
        return jnp.where(cnt >= ks, t, prefix)

    v = jax.lax.fori_loop(0, 31, step, jnp.zeros((_RG, 1), jnp.int32))

    kk = keys_ref[...]
    gt = kk > v
    count_gt = jnp.sum(gt.astype(jnp.int32), axis=1, keepdims=True)
    vf = jax.lax.bitcast_convert_type(v, jnp.float32)
    sum_gt = jnp.sum(
        jnp.where(gt, jax.lax.bitcast_convert_type(kk, jnp.float32), 0.0),
        axis=1, keepdims=True)
    s_sel = sum_gt + (num_sel - count_gt).astype(jnp.float32) * vf
    s_neg_row = jnp.where(num_sel == num_neg, sum_neg_row, s_sel)

    acc_ref[0] += sum_pos_ce + jnp.sum(s_neg_row)
    acc_ref[1] += jnp.sum(npc).astype(jnp.float32)
    out_ref[0, 0] = acc_ref[0] / acc_ref[1]


@functools.partial(jax.jit, static_argnames=("interpret",))
def _run(d, targets, interpret=False):
    grid = (_B // _RG,)
    out = pl.pallas_call(
        _body,
        grid=grid,
        in_specs=[
            pl.BlockSpec((_RG, _N), lambda g: (g, 0)),
            pl.BlockSpec((_RG, _N), lambda g: (g, 0)),
        ],
        out_specs=pl.BlockSpec(memory_space=pltpu.SMEM),
        out_shape=jax.ShapeDtypeStruct((1, 1), jnp.float32),
        scratch_shapes=[
            pltpu.VMEM((_RG, _N), jnp.int32),
            pltpu.SMEM((2,), jnp.float32),
        ],
        interpret=interpret,
    )(d, targets)
    return out[0, 0]


def kernel(predictions, targets):
    d = predictions[:, :, 1] - predictions[:, :, 0]
    return _run(d, targets)


# speedup vs baseline: 93.1059x; 3.2886x over previous
import functools

import jax
import jax.numpy as jnp
from jax.experimental import pallas as pl
from jax.experimental.pallas import tpu as pltpu

_B = 64
_N = 20000
_RG = 8
_RATIO = 3


def _body(d_ref, t_ref, out_ref, keys_ref, acc_ref):
    g = pl.program_id(0)

    @pl.when(g == 0)
    def _init():
        acc_ref[0] = 0.0
        acc_ref[1] = 0.0

    d = d_ref[...]
    lab = t_ref[...]
    pos = lab > 0

    loss = jnp.maximum(d, 0.0) + jnp.log1p(jnp.exp(-jnp.abs(d)))
    ce_pos = loss - d

    num_pos = jnp.sum(pos.astype(jnp.int32), axis=1, keepdims=True)
    sum_pos_ce = jnp.sum(jnp.where(pos, ce_pos, 0.0))
    sum_neg_row = jnp.sum(jnp.where(pos, 0.0, loss), axis=1, keepdims=True)

    npc = jnp.maximum(num_pos, 1)
    num_neg = _N - num_pos
    num_sel = jnp.minimum(npc * _RATIO, num_neg)

    def _slow():
        keys_ref[...] = jnp.where(
            pos, jnp.int32(-1), jax.lax.bitcast_convert_type(loss, jnp.int32))
        ks = jnp.maximum(num_sel, 1)

        def step(i, prefix):
            t = prefix | (jnp.int32(1) << (30 - i))
            cnt = jnp.sum((keys_ref[...] >= t).astype(jnp.int32), axis=1,
                          keepdims=True)
            return jnp.where(cnt >= ks, t, prefix)

        v = jax.lax.fori_loop(0, 31, step, jnp.zeros((_RG, 1), jnp.int32))

        kk = keys_ref[...]
        gt = kk > v
        count_gt = jnp.sum(gt.astype(jnp.int32), axis=1, keepdims=True)
        vf = jax.lax.bitcast_convert_type(v, jnp.float32)
        sum_gt = jnp.sum(
            jnp.where(gt, jax.lax.bitcast_convert_type(kk, jnp.float32), 0.0),
            axis=1, keepdims=True)
        s_sel = sum_gt + (num_sel - count_gt).astype(jnp.float32) * vf
        return jnp.where(num_sel == num_neg, sum_neg_row, s_sel)

    s_neg_row = jax.lax.cond(jnp.any(num_sel != num_neg),
                             _slow, lambda: sum_neg_row)

    acc_ref[0] += sum_pos_ce + jnp.sum(s_neg_row)
    acc_ref[1] += jnp.sum(npc).astype(jnp.float32)
    out_ref[0, 0] = acc_ref[0] / acc_ref[1]


@functools.partial(jax.jit, static_argnames=("interpret",))
def _run(d, targets, interpret=False):
    grid = (_B // _RG,)
    out = pl.pallas_call(
        _body,
        grid=grid,
        in_specs=[
            pl.BlockSpec((_RG, _N), lambda g: (g, 0)),
            pl.BlockSpec((_RG, _N), lambda g: (g, 0)),
        ],
        out_specs=pl.BlockSpec(memory_space=pltpu.SMEM),
        out_shape=jax.ShapeDtypeStruct((1, 1), jnp.float32),
        scratch_shapes=[
            pltpu.VMEM((_RG, _N), jnp.int32),
            pltpu.SMEM((2,), jnp.float32),
        ],
        interpret=interpret,
    )(d, targets)
    return out[0, 0]


def kernel(predictions, targets):
    d = predictions[:, :, 1] - predictions[:, :, 0]
    return _run(d, targets)


# speedup vs baseline: 119.4384x; 1.2828x over previous
import functools

import jax
import jax.numpy as jnp
from jax.experimental import pallas as pl
from jax.experimental.pallas import tpu as pltpu

_B = 64
_N = 20000
_RG = 8
_RATIO = 3


def _body(d_ref, t_ref, out_ref, keys_ref, acc_ref):
    g = pl.program_id(0)

    @pl.when(g == 0)
    def _init():
        acc_ref[0] = 0.0
        acc_ref[1] = 0.0

    d = d_ref[...]
    lab = t_ref[...]
    pos = lab > 0

    loss = jnp.maximum(d, 0.0) + jnp.log1p(jnp.exp(-jnp.abs(d)))

    num_pos = jnp.sum(pos.astype(jnp.int32), axis=1, keepdims=True)
    sum_all = jnp.sum(loss) - jnp.sum(jnp.where(pos, d, 0.0))

    npc = jnp.maximum(num_pos, 1)
    num_neg = _N - num_pos
    num_sel = jnp.minimum(npc * _RATIO, num_neg)

    def _slow():
        sum_neg_row = jnp.sum(jnp.where(pos, 0.0, loss), axis=1, keepdims=True)
        keys_ref[...] = jnp.where(
            pos, jnp.int32(-1), jax.lax.bitcast_convert_type(loss, jnp.int32))
        ks = jnp.maximum(num_sel, 1)

        def step(i, prefix):
            t = prefix | (jnp.int32(1) << (30 - i))
            cnt = jnp.sum((keys_ref[...] >= t).astype(jnp.int32), axis=1,
                          keepdims=True)
            return jnp.where(cnt >= ks, t, prefix)

        v = jax.lax.fori_loop(0, 31, step, jnp.zeros((_RG, 1), jnp.int32))

        kk = keys_ref[...]
        gt = kk > v
        count_gt = jnp.sum(gt.astype(jnp.int32), axis=1, keepdims=True)
        vf = jax.lax.bitcast_convert_type(v, jnp.float32)
        sum_gt = jnp.sum(
            jnp.where(gt, jax.lax.bitcast_convert_type(kk, jnp.float32), 0.0),
            axis=1, keepdims=True)
        s_sel = sum_gt + (num_sel - count_gt).astype(jnp.float32) * vf
        s_neg_row = jnp.where(num_sel == num_neg, sum_neg_row, s_sel)
        return sum_all + jnp.sum(s_neg_row - sum_neg_row)

    contrib = jax.lax.cond(jnp.any(num_sel != num_neg),
                           _slow, lambda: sum_all)

    acc_ref[0] += contrib
    acc_ref[1] += jnp.sum(npc).astype(jnp.float32)
    out_ref[0, 0] = acc_ref[0] / acc_ref[1]


@functools.partial(jax.jit, static_argnames=("interpret",))
def _run(d, targets, interpret=False):
    grid = (_B // _RG,)
    out = pl.pallas_call(
        _body,
        grid=grid,
        in_specs=[
            pl.BlockSpec((_RG, _N), lambda g: (g, 0)),
            pl.BlockSpec((_RG, _N), lambda g: (g, 0)),
        ],
        out_specs=pl.BlockSpec(memory_space=pltpu.SMEM),
        out_shape=jax.ShapeDtypeStruct((1, 1), jnp.float32),
        scratch_shapes=[
            pltpu.VMEM((_RG, _N), jnp.int32),
            pltpu.SMEM((2,), jnp.float32),
        ],
        interpret=interpret,
    )(d, targets)
    return out[0, 0]


def kernel(predictions, targets):
    d = jax.lax.dot_general(predictions, jnp.array([-1.0, 1.0], jnp.float32),
                            (((2,), (0,)), ((), ())),
                            preferred_element_type=jnp.float32)
    return _run(d, targets)
